# baseline (speedup 1.0000x reference)
import functools

import jax
import jax.numpy as jnp
from jax import lax
from jax.experimental import pallas as pl
from jax.experimental.pallas import tpu as pltpu
from jax.experimental.pallas import tpu_sc as plsc

BATCH = 16384
EMBED_DIM = 32

_info = plsc.get_sparse_core_info()
_NC, _NS = _info.num_cores, _info.num_subcores
_NW = _NC * _NS
_B_PER_W = BATCH // _NW
_CHUNK = 256
_GRP = 8


def _gather_body(user_tab, movie_tab, user_idx, movie_idx, u_out, m_out,
                 idx_uv, idx_mv, idx_shu, idx_shm, idx_us, idx_ms,
                 rows_u, rows_m, sem_u, sem_m):
    wid = lax.axis_index("s") * _NC + lax.axis_index("c")
    base = wid * _B_PER_W
    pltpu.sync_copy(user_idx.at[pl.ds(base, _B_PER_W)], idx_uv)
    pltpu.sync_copy(movie_idx.at[pl.ds(base, _B_PER_W)], idx_mv)
    pltpu.sync_copy(idx_uv, idx_shu.at[wid])
    pltpu.sync_copy(idx_mv, idx_shm.at[wid])
    pltpu.sync_copy(idx_shu.at[wid], idx_us)
    pltpu.sync_copy(idx_shm.at[wid], idx_ms)

    def fire_grp(g, off):
        for j in range(_GRP):
            i = g * _GRP + j
            pltpu.async_copy(user_tab.at[pl.ds(idx_us[off + i], 1)],
                             rows_u.at[pl.ds(i, 1)], sem_u)
            pltpu.async_copy(movie_tab.at[pl.ds(idx_ms[off + i], 1)],
                             rows_m.at[pl.ds(i, 1)], sem_m)

    def drain_grp():
        for _ in range(_GRP):
            pltpu.make_async_copy(user_tab.at[pl.ds(0, 1)],
                                  rows_u.at[pl.ds(0, 1)], sem_u).wait()
            pltpu.make_async_copy(movie_tab.at[pl.ds(0, 1)],
                                  rows_m.at[pl.ds(0, 1)], sem_m).wait()

    n_grp = _CHUNK // _GRP
    for c in range(_B_PER_W // _CHUNK):
        off = c * _CHUNK
        fire_grp(0, off)

        def step(g, carry):
            fire_grp(g, off)
            drain_grp()
            return carry

        lax.fori_loop(1, n_grp, step, 0)
        drain_grp()
        pltpu.sync_copy(rows_u, u_out.at[pl.ds(base + off, _CHUNK)])
        pltpu.sync_copy(rows_m, m_out.at[pl.ds(base + off, _CHUNK)])


_sc_gather = functools.partial(
    pl.kernel,
    out_type=(
        jax.ShapeDtypeStruct((BATCH, EMBED_DIM), jnp.float32),
        jax.ShapeDtypeStruct((BATCH, EMBED_DIM), jnp.float32),
    ),
    mesh=plsc.VectorSubcoreMesh(core_axis_name="c", subcore_axis_name="s"),
    scratch_types=[
        pltpu.VMEM((_B_PER_W,), jnp.int32),
        pltpu.VMEM((_B_PER_W,), jnp.int32),
        pltpu.VMEM_SHARED((_NW, _B_PER_W), jnp.int32),
        pltpu.VMEM_SHARED((_NW, _B_PER_W), jnp.int32),
        pltpu.SMEM((_B_PER_W,), jnp.int32),
        pltpu.SMEM((_B_PER_W,), jnp.int32),
        pltpu.VMEM((_CHUNK, EMBED_DIM), jnp.float32),
        pltpu.VMEM((_CHUNK, EMBED_DIM), jnp.float32),
        pltpu.SemaphoreType.DMA,
        pltpu.SemaphoreType.DMA,
    ],
    compiler_params=pltpu.CompilerParams(use_tc_tiling_on_sc=True),
)(_gather_body)


_TC = 2048


def _transpose_body(src_ref, dst_ref):
    dst_ref[:] = jnp.swapaxes(src_ref[:], 0, 1)


def _transpose(tabT, n_rows):
    return pl.pallas_call(
        _transpose_body,
        grid=(pl.cdiv(n_rows, _TC),),
        in_specs=[pl.BlockSpec((EMBED_DIM, _TC), lambda i: (0, i))],
        out_specs=pl.BlockSpec((_TC, EMBED_DIM), lambda i: (i, 0)),
        out_shape=jax.ShapeDtypeStruct((n_rows, EMBED_DIM), jnp.float32),
    )(tabT)


_BLK = 1024


def _mlp_body(u_ref, m_ref, w1u_ref, w1m_ref, b1_ref, w2_ref, b2_ref,
              w3_ref, b3_ref, out_ref):
    h = jnp.dot(u_ref[:], w1u_ref[:], preferred_element_type=jnp.float32)
    h += jnp.dot(m_ref[:], w1m_ref[:], preferred_element_type=jnp.float32)
    h = jnp.maximum(h + b1_ref[:], 0.0)
    h = jnp.dot(h, w2_ref[:], preferred_element_type=jnp.float32)
    h = jnp.maximum(h + b2_ref[:], 0.0)
    out_ref[:] = jnp.sum(h * w3_ref[:], axis=1) + b3_ref[0, 0]


def _mlp(u, m, W1, b1, W2, b2, W3, b3):
    grid = (BATCH // _BLK,)
    full = lambda i: (0, 0)
    return pl.pallas_call(
        _mlp_body,
        grid=grid,
        in_specs=[
            pl.BlockSpec((_BLK, EMBED_DIM), lambda i: (i, 0)),
            pl.BlockSpec((_BLK, EMBED_DIM), lambda i: (i, 0)),
            pl.BlockSpec((EMBED_DIM, 64), full),
            pl.BlockSpec((EMBED_DIM, 64), full),
            pl.BlockSpec((1, 64), full),
            pl.BlockSpec((64, 32), full),
            pl.BlockSpec((1, 32), full),
            pl.BlockSpec((1, 32), full),
            pl.BlockSpec((1, 1), full),
        ],
        out_specs=pl.BlockSpec((_BLK,), lambda i: (i,)),
        out_shape=jax.ShapeDtypeStruct((BATCH,), jnp.float32),
    )(u, m, W1[:EMBED_DIM], W1[EMBED_DIM:], b1.reshape(1, 64), W2,
      b2.reshape(1, 32), W3.reshape(1, 32), b3.reshape(1, 1))


@jax.jit
def kernel(user, movie, user_table, movie_table, W1, b1, W2, b2, W3, b3):
    ut = _transpose(user_table.T, 1000000)
    mt = _transpose(movie_table.T, 100000)
    u_rows, m_rows = _sc_gather(ut, mt, user.astype(jnp.int32),
                                movie.astype(jnp.int32))
    return _mlp(u_rows, m_rows, W1, b1, W2, b2, W3, b3)


# speedup vs baseline: 1.4622x; 1.4622x over previous
import functools

import jax
import jax.numpy as jnp
from jax import lax
from jax.experimental import pallas as pl
from jax.experimental.pallas import tpu as pltpu
from jax.experimental.pallas import tpu_sc as plsc

BATCH = 16384
EMBED_DIM = 32

_info = plsc.get_sparse_core_info()
_NC, _NS = _info.num_cores, _info.num_subcores
_NW = _NC * _NS
_B_PER_W = BATCH // _NW
_CHUNK = 256
_GRP = 8


def _gather_body(tab, idx_hbm, out, idx_v, idx_sh, idx_s, rows, sem):
    wid = lax.axis_index("s") * _NC + lax.axis_index("c")
    base = wid * _B_PER_W
    pltpu.sync_copy(idx_hbm.at[pl.ds(base, _B_PER_W)], idx_v)
    pltpu.sync_copy(idx_v, idx_sh.at[wid])
    pltpu.sync_copy(idx_sh.at[wid], idx_s)

    def fire_grp(g, off):
        for j in range(_GRP):
            i = g * _GRP + j
            pltpu.async_copy(tab.at[pl.ds(idx_s[off + i], 1)],
                             rows.at[pl.ds(i, 1)], sem)

    def drain_grp():
        for _ in range(_GRP):
            pltpu.make_async_copy(tab.at[pl.ds(0, 1)],
                                  rows.at[pl.ds(0, 1)], sem).wait()

    n_grp = _CHUNK // _GRP
    for c in range(_B_PER_W // _CHUNK):
        off = c * _CHUNK
        fire_grp(0, off)
        fire_grp(1, off)

        def step(g, carry):
            fire_grp(g, off)
            drain_grp()
            return carry

        lax.fori_loop(2, n_grp, step, 0)
        drain_grp()
        drain_grp()
        pltpu.sync_copy(rows, out.at[pl.ds(base + off, _CHUNK)])


def _make_gather():
    return functools.partial(
        pl.kernel,
        out_type=jax.ShapeDtypeStruct((BATCH, EMBED_DIM), jnp.float32),
        mesh=plsc.VectorSubcoreMesh(core_axis_name="c",
                                    subcore_axis_name="s"),
        scratch_types=[
            pltpu.VMEM((_B_PER_W,), jnp.int32),
            pltpu.VMEM_SHARED((_NW, _B_PER_W), jnp.int32),
            pltpu.SMEM((_B_PER_W,), jnp.int32),
            pltpu.VMEM((_CHUNK, EMBED_DIM), jnp.float32),
            pltpu.SemaphoreType.DMA,
        ],
        compiler_params=pltpu.CompilerParams(use_tc_tiling_on_sc=True),
    )(_gather_body)


_gather_u = _make_gather()
_gather_m = _make_gather()


_BLK = 1024


def _mlp_body(u_ref, m_ref, w1u_ref, w1m_ref, b1_ref, w2_ref, b2_ref,
              w3_ref, b3_ref, out_ref):
    h = jnp.dot(u_ref[:], w1u_ref[:], preferred_element_type=jnp.float32)
    h += jnp.dot(m_ref[:], w1m_ref[:], preferred_element_type=jnp.float32)
    h = jnp.maximum(h + b1_ref[:], 0.0)
    h = jnp.dot(h, w2_ref[:], preferred_element_type=jnp.float32)
    h = jnp.maximum(h + b2_ref[:], 0.0)
    out_ref[:] = jnp.sum(h * w3_ref[:], axis=1) + b3_ref[0, 0]


def _mlp(u, m, W1, b1, W2, b2, W3, b3):
    grid = (BATCH // _BLK,)
    full = lambda i: (0, 0)
    return pl.pallas_call(
        _mlp_body,
        grid=grid,
        in_specs=[
            pl.BlockSpec((_BLK, EMBED_DIM), lambda i: (i, 0)),
            pl.BlockSpec((_BLK, EMBED_DIM), lambda i: (i, 0)),
            pl.BlockSpec((EMBED_DIM, 64), full),
            pl.BlockSpec((EMBED_DIM, 64), full),
            pl.BlockSpec((1, 64), full),
            pl.BlockSpec((64, 32), full),
            pl.BlockSpec((1, 32), full),
            pl.BlockSpec((1, 32), full),
            pl.BlockSpec((1, 1), full),
        ],
        out_specs=pl.BlockSpec((_BLK,), lambda i: (i,)),
        out_shape=jax.ShapeDtypeStruct((BATCH,), jnp.float32),
    )(u, m, W1[:EMBED_DIM], W1[EMBED_DIM:], b1.reshape(1, 64), W2,
      b2.reshape(1, 32), W3.reshape(1, 32), b3.reshape(1, 1))


@jax.jit
def kernel(user, movie, user_table, movie_table, W1, b1, W2, b2, W3, b3):
    m_rows = _gather_m(movie_table, movie.astype(jnp.int32))
    u_rows = _gather_u(user_table, user.astype(jnp.int32))
    return _mlp(u_rows, m_rows, W1, b1, W2, b2, W3, b3)
